# Initial kernel scaffold; baseline (speedup 1.0000x reference)
import jax
import jax.numpy as jnp
from jax.experimental import pallas as pl


def kernel(source_node, target_node, edge_index, edge_attr, distance, W_msg, b_msg, W_res, W_comb, b_comb, ln_gamma, ln_beta):
    raise NotImplementedError("write your pallas kernel here")


# speedup vs baseline: 3.8136x; 3.8136x over previous
import jax
import jax.numpy as jnp
from jax import lax
from jax.experimental import pallas as pl
from jax.experimental.pallas import tpu as pltpu
from jax.experimental.pallas import tpu_sc as plsc

N = 10000
E = 320000
D = 128
HID = 128
OUT = 128

NC = 2
NS = 16
NW = NC * NS
EPW = E // NW
CHUNK = 40
NCHUNK = EPW // CHUNK
HALF = NCHUNK // 2
ACC_N = 10240
ROWS_PER_SUB = ACC_N // NS
SI_GROUPS = (0, 16, 24)
CGROUPS = ((0, 16), (16, 16), (32, 8))

ROW_BLK = 400


def _pre_body(src_ref, tgt_ref, wst_ref, wtt_ref, b_ref, s_ref, t_ref):
    s_ref[...] = (
        jnp.dot(src_ref[...], wst_ref[...], preferred_element_type=jnp.float32)
        + b_ref[...]
    )
    t_ref[...] = jnp.dot(
        tgt_ref[...], wtt_ref[...], preferred_element_type=jnp.float32
    )


def _post_body(tgt_ref, s0_ref, s1_ref, c0_ref, c1_ref, wrt_ref, wc1t_ref,
               wc2t_ref, bc_ref, g_ref, bt_ref, out_ref):
    cnt = jnp.maximum(c0_ref[:, 0:1] + c1_ref[:, 0:1], 1.0)
    aggr = (s0_ref[...] + s1_ref[...]) / cnt
    w1 = wrt_ref[...] + wc1t_ref[...]
    h = (
        jnp.dot(tgt_ref[...], w1, preferred_element_type=jnp.float32)
        + jnp.dot(aggr, wc2t_ref[...], preferred_element_type=jnp.float32)
        + bc_ref[...]
    )
    mean = jnp.mean(h, axis=-1, keepdims=True)
    var = jnp.mean(jnp.square(h - mean), axis=-1, keepdims=True)
    out_ref[...] = (h - mean) / jnp.sqrt(var + 1e-5) * g_ref[...] + bt_ref[...]


def _sc_body(s_hbm, t_hbm, epk_hbm, dist_hbm, wd_hbm,
             sums_hbm, cnts_hbm,
             eb0, eb1, db0, db1, sg0, sg1, tg0, tg1, si0, si1,
             wd_v, ones_v,
             acc_sh, cnt_sh,
             semS0, semS1, semT0, semT1, semA0, semA1, semC0, semC1):
    c = lax.axis_index("c")
    s = lax.axis_index("s")
    wid = s * NC + c
    row0 = s * ROWS_PER_SUB
    out0 = c * ACC_N + row0

    z16 = jnp.zeros((16,), jnp.float32)
    o16 = jnp.ones((16,), jnp.float32)

    def zrow_body(r, cc):
        for j in range(8):
            sg0[r, pl.ds(16 * j, 16)] = z16
        ones_v[r, :] = z16
        return cc

    lax.fori_loop(0, CHUNK, zrow_body, 0)
    for k in range(ROWS_PER_SUB // CHUNK):
        pltpu.sync_copy(sg0, acc_sh.at[pl.ds(row0 + k * CHUNK, CHUNK)])
        pltpu.sync_copy(ones_v, cnt_sh.at[pl.ds(row0 + k * CHUNK, CHUNK)])

    def orow_body(r, cc):
        ones_v[r, :] = o16
        return cc

    lax.fori_loop(0, CHUNK, orow_body, 0)
    pltpu.sync_copy(wd_hbm, wd_v)
    plsc.subcore_barrier()

    wdj = [wd_v[pl.ds(16 * j, 16)] for j in range(8)]
    ebase = wid * EPW

    ebs = (eb0, eb1)
    dbs = (db0, db1)
    sgs = (sg0, sg1)
    tgs = (tg0, tg1)
    sis = (si0, si1)
    semS = (semS0, semS1)
    semT = (semT0, semT1)
    semA = (semA0, semA1)
    semC = (semC0, semC1)

    def load_and_fire(ci, p):
        pltpu.sync_copy(epk_hbm.at[:, pl.ds(ebase + ci * CHUNK, CHUNK)],
                        ebs[p])
        pltpu.sync_copy(dist_hbm.at[pl.ds(ebase + ci * CHUNK, CHUNK)],
                        dbs[p].at[pl.ds(0, CHUNK)])
        pltpu.async_copy(s_hbm.at[ebs[p].at[0]], sgs[p], semS[p])
        pltpu.async_copy(t_hbm.at[ebs[p].at[1]], tgs[p], semT[p])

    def wait_gathers(p):
        pltpu.make_async_copy(s_hbm.at[ebs[p].at[0]], sgs[p], semS[p]).wait()
        pltpu.make_async_copy(t_hbm.at[ebs[p].at[1]], tgs[p], semT[p]).wait()

    def wait_scatters(p):
        pltpu.make_async_copy(sgs[p], acc_sh.at[sis[p]], semA[p]).wait()
        pltpu.make_async_copy(ones_v, cnt_sh.at[sis[p]], semC[p]).wait()

    def phase(ci, k, p, prefetch, guard_sc):
        q = 1 - p
        if prefetch is not None:
            @pl.when(prefetch)
            def _():
                load_and_fire(ci + 1, q)
        else:
            load_and_fire(ci + 1, q)
        wait_gathers(p)

        def group(g0, cnt):
            dvec = dbs[p][pl.ds(g0, 16)]
            for ell in range(cnt):
                e = g0 + ell
                db = jnp.broadcast_to(dvec[ell], (16,))
                for j in range(8):
                    m = jnp.maximum(
                        sgs[p][e, pl.ds(16 * j, 16)]
                        + tgs[p][e, pl.ds(16 * j, 16)]
                        + db * wdj[j],
                        0.0,
                    )
                    sgs[p][e, pl.ds(16 * j, 16)] = m

        for g0, cnt in CGROUPS:
            group(g0, cnt)
        for v0 in SI_GROUPS:
            sis[p][pl.ds(v0, 16)] = ebs[p][1, pl.ds(v0, 16)]
        cpa = pltpu.async_copy(sgs[p], acc_sh.at[sis[p]], semA[p], add=True)
        cpc = pltpu.async_copy(ones_v, cnt_sh.at[sis[p]], semC[p], add=True)
        cpa.wait()
        cpc.wait()

    load_and_fire(0, 0)

    def pipe_body(k, carry):
        ci0 = 2 * k
        phase(ci0, k, 0, prefetch=None, guard_sc=None)
        phase(ci0 + 1, k, 1, prefetch=(k < HALF - 1), guard_sc=None)
        return carry

    lax.fori_loop(0, HALF, pipe_body, 0)
    plsc.subcore_barrier()

    for k in range(ROWS_PER_SUB // CHUNK):
        pltpu.sync_copy(acc_sh.at[pl.ds(row0 + k * CHUNK, CHUNK)], sg0)
        pltpu.sync_copy(sg0, sums_hbm.at[pl.ds(out0 + k * CHUNK, CHUNK)])
        pltpu.sync_copy(cnt_sh.at[pl.ds(row0 + k * CHUNK, CHUNK)], ones_v)
        pltpu.sync_copy(ones_v, cnts_hbm.at[pl.ds(out0 + k * CHUNK, CHUNK)])


@jax.jit
def kernel(source_node, target_node, edge_index, edge_attr, distance,
           W_msg, b_msg, W_res, W_comb, b_comb, ln_gamma, ln_beta):
    del edge_attr

    wmt = W_msg.T
    wst = wmt[:D]
    wtt = wmt[D:2 * D]
    wd = wmt[2 * D]
    b2 = b_msg.reshape(1, HID)
    wrt = W_res.T
    wct = W_comb.T
    wc1t = wct[:D]
    wc2t = wct[D:]
    bc2 = b_comb.reshape(1, OUT)
    g2 = ln_gamma.reshape(1, OUT)
    bt2 = ln_beta.reshape(1, OUT)
    epk = edge_index
    dist = distance.reshape(E)

    s_proj, t_proj = pl.pallas_call(
        _pre_body,
        grid=(N // ROW_BLK,),
        in_specs=[
            pl.BlockSpec((ROW_BLK, D), lambda i: (i, 0)),
            pl.BlockSpec((ROW_BLK, D), lambda i: (i, 0)),
            pl.BlockSpec((D, HID), lambda i: (0, 0)),
            pl.BlockSpec((D, HID), lambda i: (0, 0)),
            pl.BlockSpec((1, HID), lambda i: (0, 0)),
        ],
        out_specs=[
            pl.BlockSpec((ROW_BLK, HID), lambda i: (i, 0)),
            pl.BlockSpec((ROW_BLK, HID), lambda i: (i, 0)),
        ],
        out_shape=[
            jax.ShapeDtypeStruct((N, HID), jnp.float32),
            jax.ShapeDtypeStruct((N, HID), jnp.float32),
        ],
    )(source_node, target_node, wst, wtt, b2)

    sc_edge = pl.kernel(
        _sc_body,
        mesh=plsc.VectorSubcoreMesh(core_axis_name="c", subcore_axis_name="s"),
        compiler_params=pltpu.CompilerParams(use_tc_tiling_on_sc=False),
        out_type=[
            jax.ShapeDtypeStruct((NC * ACC_N, HID), jnp.float32),
            jax.ShapeDtypeStruct((NC * ACC_N, 16), jnp.float32),
        ],
        scratch_types=[
            pltpu.VMEM((2, CHUNK), jnp.int32),
            pltpu.VMEM((2, CHUNK), jnp.int32),
            pltpu.VMEM((CHUNK + 8,), jnp.float32),
            pltpu.VMEM((CHUNK + 8,), jnp.float32),
            pltpu.VMEM((CHUNK, HID), jnp.float32),
            pltpu.VMEM((CHUNK, HID), jnp.float32),
            pltpu.VMEM((CHUNK, HID), jnp.float32),
            pltpu.VMEM((CHUNK, HID), jnp.float32),
            pltpu.VMEM((CHUNK,), jnp.int32),
            pltpu.VMEM((CHUNK,), jnp.int32),
            pltpu.VMEM((HID,), jnp.float32),
            pltpu.VMEM((CHUNK, 16), jnp.float32),
            pltpu.VMEM_SHARED((ACC_N, HID), jnp.float32),
            pltpu.VMEM_SHARED((ACC_N, 16), jnp.float32),
            pltpu.SemaphoreType.DMA,
            pltpu.SemaphoreType.DMA,
            pltpu.SemaphoreType.DMA,
            pltpu.SemaphoreType.DMA,
            pltpu.SemaphoreType.DMA,
            pltpu.SemaphoreType.DMA,
            pltpu.SemaphoreType.DMA,
            pltpu.SemaphoreType.DMA,
        ],
    )
    sums, cnts = sc_edge(s_proj, t_proj, epk, dist, wd)

    out = pl.pallas_call(
        _post_body,
        grid=(N // ROW_BLK,),
        in_specs=[
            pl.BlockSpec((ROW_BLK, D), lambda i: (i, 0)),
            pl.BlockSpec((ROW_BLK, HID), lambda i: (i, 0)),
            pl.BlockSpec((ROW_BLK, HID), lambda i: (i, 0)),
            pl.BlockSpec((ROW_BLK, 16), lambda i: (i, 0)),
            pl.BlockSpec((ROW_BLK, 16), lambda i: (i, 0)),
            pl.BlockSpec((D, OUT), lambda i: (0, 0)),
            pl.BlockSpec((D, OUT), lambda i: (0, 0)),
            pl.BlockSpec((HID, OUT), lambda i: (0, 0)),
            pl.BlockSpec((1, OUT), lambda i: (0, 0)),
            pl.BlockSpec((1, OUT), lambda i: (0, 0)),
            pl.BlockSpec((1, OUT), lambda i: (0, 0)),
        ],
        out_specs=pl.BlockSpec((ROW_BLK, OUT), lambda i: (i, 0)),
        out_shape=jax.ShapeDtypeStruct((N, OUT), jnp.float32),
    )(target_node, sums[:N], sums[ACC_N:ACC_N + N], cnts[:N],
      cnts[ACC_N:ACC_N + N], wrt, wc1t, wc2t, bc2, g2, bt2)
    return out


# speedup vs baseline: 3.8746x; 1.0160x over previous
import jax
import jax.numpy as jnp
from jax import lax
from jax.experimental import pallas as pl
from jax.experimental.pallas import tpu as pltpu
from jax.experimental.pallas import tpu_sc as plsc

N = 10000
E = 320000
D = 128
HID = 128
OUT = 128

NC = 2
NS = 16
NW = NC * NS
EPW = E // NW
CHUNK = 40
NCHUNK = EPW // CHUNK
HALF = NCHUNK // 2
ACC_N = 10240
ROWS_PER_SUB = ACC_N // NS
SI_GROUPS = (0, 16, 24)
CGROUPS = ((0, 16), (16, 16), (32, 8))

ROW_BLK = 400


def _pre_body(src_ref, tgt_ref, wst_ref, wtt_ref, b_ref, s_ref, t_ref):
    s_ref[...] = (
        jnp.dot(src_ref[...], wst_ref[...], preferred_element_type=jnp.float32)
        + b_ref[...]
    )
    t_ref[...] = jnp.dot(
        tgt_ref[...], wtt_ref[...], preferred_element_type=jnp.float32
    )


def _post_body(tgt_ref, s0_ref, s1_ref, c0_ref, c1_ref, wrt_ref, wc1t_ref,
               wc2t_ref, bc_ref, g_ref, bt_ref, out_ref):
    cnt = jnp.maximum(c0_ref[:, 0:1] + c1_ref[:, 0:1], 1.0)
    aggr = (s0_ref[...] + s1_ref[...]) / cnt
    w1 = wrt_ref[...] + wc1t_ref[...]
    h = (
        jnp.dot(tgt_ref[...], w1, preferred_element_type=jnp.float32)
        + jnp.dot(aggr, wc2t_ref[...], preferred_element_type=jnp.float32)
        + bc_ref[...]
    )
    mean = jnp.mean(h, axis=-1, keepdims=True)
    var = jnp.mean(jnp.square(h - mean), axis=-1, keepdims=True)
    out_ref[...] = (h - mean) / jnp.sqrt(var + 1e-5) * g_ref[...] + bt_ref[...]


def _sc_body(s_hbm, t_hbm, epk_hbm, dist_hbm, wd_hbm,
             sums_hbm, cnts_hbm,
             eb0, eb1, db0, db1, sg0, sg1, tg0, tg1, si0, si1,
             wd_v, ones_v,
             acc_sh, cnt_sh,
             semS0, semS1, semT0, semT1, semA0, semA1, semC0, semC1):
    c = lax.axis_index("c")
    s = lax.axis_index("s")
    wid = s * NC + c
    row0 = s * ROWS_PER_SUB
    out0 = c * ACC_N + row0

    z16 = jnp.zeros((16,), jnp.float32)
    o16 = jnp.ones((16,), jnp.float32)

    def zrow_body(r, cc):
        for j in range(8):
            sg0[r, pl.ds(16 * j, 16)] = z16
        ones_v[r, :] = z16
        return cc

    lax.fori_loop(0, CHUNK, zrow_body, 0)
    for k in range(ROWS_PER_SUB // CHUNK):
        pltpu.sync_copy(sg0, acc_sh.at[pl.ds(row0 + k * CHUNK, CHUNK)])
        pltpu.sync_copy(ones_v, cnt_sh.at[pl.ds(row0 + k * CHUNK, CHUNK)])

    def orow_body(r, cc):
        ones_v[r, :] = o16
        return cc

    lax.fori_loop(0, CHUNK, orow_body, 0)
    pltpu.sync_copy(wd_hbm, wd_v)
    plsc.subcore_barrier()

    wdj = [wd_v[pl.ds(16 * j, 16)] for j in range(8)]
    ebase = wid * EPW

    ebs = (eb0, eb1)
    dbs = (db0, db1)
    sgs = (sg0, sg1)
    tgs = (tg0, tg1)
    sis = (si0, si1)
    semS = (semS0, semS1)
    semT = (semT0, semT1)
    semA = (semA0, semA1)
    semC = (semC0, semC1)

    def load_and_fire(ci, p):
        pltpu.sync_copy(epk_hbm.at[:, pl.ds(ebase + ci * CHUNK, CHUNK)],
                        ebs[p])
        pltpu.sync_copy(dist_hbm.at[pl.ds(ebase + ci * CHUNK, CHUNK)],
                        dbs[p].at[pl.ds(0, CHUNK)])
        pltpu.async_copy(s_hbm.at[ebs[p].at[0]], sgs[p], semS[p])
        pltpu.async_copy(t_hbm.at[ebs[p].at[1]], tgs[p], semT[p])

    def wait_gathers(p):
        pltpu.make_async_copy(s_hbm.at[ebs[p].at[0]], sgs[p], semS[p]).wait()
        pltpu.make_async_copy(t_hbm.at[ebs[p].at[1]], tgs[p], semT[p]).wait()

    def wait_scatters(p):
        pltpu.make_async_copy(sgs[p], acc_sh.at[sis[p]], semA[p]).wait()
        pltpu.make_async_copy(ones_v, cnt_sh.at[sis[p]], semC[p]).wait()

    def phase(ci, k, p, prefetch, guard_sc):
        q = 1 - p

        def prefetch_blk():
            if guard_sc is not None:
                @pl.when(guard_sc)
                def _():
                    wait_scatters(q)
            else:
                wait_scatters(q)
            load_and_fire(ci + 1, q)

        if prefetch is not None:
            @pl.when(prefetch)
            def _():
                prefetch_blk()
        else:
            prefetch_blk()
        wait_gathers(p)

        def group(g0, cnt):
            dvec = dbs[p][pl.ds(g0, 16)]
            for ell in range(cnt):
                e = g0 + ell
                db = jnp.broadcast_to(dvec[ell], (16,))
                for j in range(8):
                    m = jnp.maximum(
                        sgs[p][e, pl.ds(16 * j, 16)]
                        + tgs[p][e, pl.ds(16 * j, 16)]
                        + db * wdj[j],
                        0.0,
                    )
                    sgs[p][e, pl.ds(16 * j, 16)] = m

        for g0, cnt in CGROUPS:
            group(g0, cnt)
        for v0 in SI_GROUPS:
            sis[p][pl.ds(v0, 16)] = ebs[p][1, pl.ds(v0, 16)]
        pltpu.async_copy(sgs[p], acc_sh.at[sis[p]], semA[p], add=True)
        pltpu.async_copy(ones_v, cnt_sh.at[sis[p]], semC[p], add=True)

    load_and_fire(0, 0)

    def pipe_body(k, carry):
        ci0 = 2 * k
        phase(ci0, k, 0, prefetch=None, guard_sc=(k >= 1))
        phase(ci0 + 1, k, 1, prefetch=(k < HALF - 1), guard_sc=None)
        return carry

    lax.fori_loop(0, HALF, pipe_body, 0)
    wait_scatters(0)
    wait_scatters(1)
    plsc.subcore_barrier()

    for k in range(ROWS_PER_SUB // CHUNK):
        pltpu.sync_copy(acc_sh.at[pl.ds(row0 + k * CHUNK, CHUNK)], sg0)
        pltpu.sync_copy(sg0, sums_hbm.at[pl.ds(out0 + k * CHUNK, CHUNK)])
        pltpu.sync_copy(cnt_sh.at[pl.ds(row0 + k * CHUNK, CHUNK)], ones_v)
        pltpu.sync_copy(ones_v, cnts_hbm.at[pl.ds(out0 + k * CHUNK, CHUNK)])


@jax.jit
def kernel(source_node, target_node, edge_index, edge_attr, distance,
           W_msg, b_msg, W_res, W_comb, b_comb, ln_gamma, ln_beta):
    del edge_attr

    wmt = W_msg.T
    wst = wmt[:D]
    wtt = wmt[D:2 * D]
    wd = wmt[2 * D]
    b2 = b_msg.reshape(1, HID)
    wrt = W_res.T
    wct = W_comb.T
    wc1t = wct[:D]
    wc2t = wct[D:]
    bc2 = b_comb.reshape(1, OUT)
    g2 = ln_gamma.reshape(1, OUT)
    bt2 = ln_beta.reshape(1, OUT)
    epk = edge_index
    dist = distance.reshape(E)

    s_proj, t_proj = pl.pallas_call(
        _pre_body,
        grid=(N // ROW_BLK,),
        in_specs=[
            pl.BlockSpec((ROW_BLK, D), lambda i: (i, 0)),
            pl.BlockSpec((ROW_BLK, D), lambda i: (i, 0)),
            pl.BlockSpec((D, HID), lambda i: (0, 0)),
            pl.BlockSpec((D, HID), lambda i: (0, 0)),
            pl.BlockSpec((1, HID), lambda i: (0, 0)),
        ],
        out_specs=[
            pl.BlockSpec((ROW_BLK, HID), lambda i: (i, 0)),
            pl.BlockSpec((ROW_BLK, HID), lambda i: (i, 0)),
        ],
        out_shape=[
            jax.ShapeDtypeStruct((N, HID), jnp.float32),
            jax.ShapeDtypeStruct((N, HID), jnp.float32),
        ],
    )(source_node, target_node, wst, wtt, b2)

    sc_edge = pl.kernel(
        _sc_body,
        mesh=plsc.VectorSubcoreMesh(core_axis_name="c", subcore_axis_name="s"),
        compiler_params=pltpu.CompilerParams(use_tc_tiling_on_sc=False),
        out_type=[
            jax.ShapeDtypeStruct((NC * ACC_N, HID), jnp.float32),
            jax.ShapeDtypeStruct((NC * ACC_N, 16), jnp.float32),
        ],
        scratch_types=[
            pltpu.VMEM((2, CHUNK), jnp.int32),
            pltpu.VMEM((2, CHUNK), jnp.int32),
            pltpu.VMEM((CHUNK + 8,), jnp.float32),
            pltpu.VMEM((CHUNK + 8,), jnp.float32),
            pltpu.VMEM((CHUNK, HID), jnp.float32),
            pltpu.VMEM((CHUNK, HID), jnp.float32),
            pltpu.VMEM((CHUNK, HID), jnp.float32),
            pltpu.VMEM((CHUNK, HID), jnp.float32),
            pltpu.VMEM((CHUNK,), jnp.int32),
            pltpu.VMEM((CHUNK,), jnp.int32),
            pltpu.VMEM((HID,), jnp.float32),
            pltpu.VMEM((CHUNK, 16), jnp.float32),
            pltpu.VMEM_SHARED((ACC_N, HID), jnp.float32),
            pltpu.VMEM_SHARED((ACC_N, 16), jnp.float32),
            pltpu.SemaphoreType.DMA,
            pltpu.SemaphoreType.DMA,
            pltpu.SemaphoreType.DMA,
            pltpu.SemaphoreType.DMA,
            pltpu.SemaphoreType.DMA,
            pltpu.SemaphoreType.DMA,
            pltpu.SemaphoreType.DMA,
            pltpu.SemaphoreType.DMA,
        ],
    )
    sums, cnts = sc_edge(s_proj, t_proj, epk, dist, wd)

    out = pl.pallas_call(
        _post_body,
        grid=(N // ROW_BLK,),
        in_specs=[
            pl.BlockSpec((ROW_BLK, D), lambda i: (i, 0)),
            pl.BlockSpec((ROW_BLK, HID), lambda i: (i, 0)),
            pl.BlockSpec((ROW_BLK, HID), lambda i: (i, 0)),
            pl.BlockSpec((ROW_BLK, 16), lambda i: (i, 0)),
            pl.BlockSpec((ROW_BLK, 16), lambda i: (i, 0)),
            pl.BlockSpec((D, OUT), lambda i: (0, 0)),
            pl.BlockSpec((D, OUT), lambda i: (0, 0)),
            pl.BlockSpec((HID, OUT), lambda i: (0, 0)),
            pl.BlockSpec((1, OUT), lambda i: (0, 0)),
            pl.BlockSpec((1, OUT), lambda i: (0, 0)),
            pl.BlockSpec((1, OUT), lambda i: (0, 0)),
        ],
        out_specs=pl.BlockSpec((ROW_BLK, OUT), lambda i: (i, 0)),
        out_shape=jax.ShapeDtypeStruct((N, OUT), jnp.float32),
    )(target_node, sums[:N], sums[ACC_N:ACC_N + N], cnts[:N],
      cnts[ACC_N:ACC_N + N], wrt, wc1t, wc2t, bc2, g2, bt2)
    return out
